# baseline (speedup 1.0000x reference)
import functools

import jax
import jax.numpy as jnp
from jax import lax
from jax.experimental import pallas as pl
from jax.experimental.pallas import tpu as pltpu
from jax.experimental.pallas import tpu_sc as plsc

VOCAB = 100000
EMBED_DIM = 128
ROWS, SEQ = 4096, 50

NUM_CORES = 2
NUM_SUBCORES = 16
NW = NUM_CORES * NUM_SUBCORES
R_PER_W = ROWS // NW

PAIR = 2
P_PER_W = R_PER_W // PAIR
NBUF = 8
LAG = 2
N_GROUPS = P_PER_W // NBUF


def _emb_kernel(table_hbm, tok_hbm, out_hbm, idx_v, rows_v, gsems, ssems):
    wid = lax.axis_index("s") * NUM_CORES + lax.axis_index("c")
    base = wid * R_PER_W
    pltpu.sync_copy(tok_hbm.at[pl.ds(base, R_PER_W)], idx_v)

    class _Pair:
        def __init__(self, copies):
            self.copies = copies

        def start(self):
            for c in self.copies:
                c.start()

        def wait(self):
            for c in self.copies:
                c.wait()

    def gather(p, b):
        return _Pair([
            pltpu.make_async_copy(
                table_hbm.at[idx_v.at[p * PAIR + k]],
                rows_v.at[b, k],
                gsems.at[b],
            )
            for k in range(PAIR)
        ])

    def scatter(p, b):
        return pltpu.make_async_copy(
            rows_v.at[b],
            out_hbm.at[pl.ds(base + p * PAIR, PAIR)],
            ssems.at[b],
        )

    for b in range(NBUF):
        gather(b, b).start()

    def step(p, b, pl_, bl, do_lag):
        gather(p, b).wait()
        scatter(p, b).start()
        if do_lag:
            scatter(pl_, bl).wait()
            gather(pl_ + NBUF, bl).start()

    for b in range(NBUF):
        step(b, b, b - LAG, (b - LAG) % NBUF, b >= LAG)

    def group_body(gi, carry):
        p0 = gi * NBUF
        for b in range(NBUF):
            step(p0 + b, b, p0 + b - LAG, (b - LAG) % NBUF, True)
        return carry

    lax.fori_loop(1, N_GROUPS - 1, group_body, 0)

    p0 = (N_GROUPS - 1) * NBUF
    for b in range(NBUF):
        p = p0 + b
        gather(p, b).wait()
        scatter(p, b).start()
        pl_, bl = p - LAG, (b - LAG) % NBUF
        scatter(pl_, bl).wait()
        if pl_ + NBUF < P_PER_W:
            gather(pl_ + NBUF, bl).start()
    for k in range(LAG):
        p = P_PER_W - LAG + k
        scatter(p, p % NBUF).wait()


@functools.partial(jax.jit)
def _embedding_lookup(table, tokens):
    mesh = plsc.VectorSubcoreMesh(core_axis_name="c", subcore_axis_name="s")
    return pl.kernel(
        _emb_kernel,
        out_type=jax.ShapeDtypeStruct((ROWS, SEQ, EMBED_DIM), jnp.float32),
        mesh=mesh,
        scratch_types=[
            pltpu.VMEM((R_PER_W, SEQ), jnp.int32),
            pltpu.VMEM((NBUF, PAIR, SEQ, EMBED_DIM), jnp.float32),
            pltpu.SemaphoreType.DMA((NBUF,)),
            pltpu.SemaphoreType.DMA((NBUF,)),
        ],
    )(table, tokens)


def kernel(tokens, token_embedding):
    return _embedding_lookup(token_embedding, tokens)
